# Initial kernel scaffold; baseline (speedup 1.0000x reference)
import jax
import jax.numpy as jnp
from jax.experimental import pallas as pl


def kernel(x, edge_index, edge_pe, W1, b1, W2, b2, W3, b3, W4, b4):
    raise NotImplementedError("write your pallas kernel here")


# speedup vs baseline: 1.2599x; 1.2599x over previous
import jax
import jax.numpy as jnp
from jax import lax
from jax.experimental import pallas as pl
from jax.experimental.pallas import tpu as pltpu
from jax.experimental.pallas import tpu_sc as plsc

N_NODES = 10000
E = 320000
D = 128
PE_DIM = 16
GW = 64
ACC_W = 80
H_ROWS = 10112
ROWS_PER_TILE = H_ROWS // 16
CHUNK = 128
E_PAD = 327680
NCHUNK_TILE = E_PAD // CHUNK // 16


def _g_kernel(x_ref, w1a_ref, o_ref):
    o_ref[0] = jnp.dot(x_ref[...], w1a_ref[0],
                       preferred_element_type=jnp.float32)


def _p_kernel(pe_ref, w1b_ref, b1_ref, o_ref):
    o_ref[0] = jnp.dot(pe_ref[...], w1b_ref[0],
                       preferred_element_type=jnp.float32) + b1_ref[0]


def _edge_kernel(g_hbm, p_hbm, src_hbm, dst_hbm, out_hbm,
                 src_idx, dst_idx, gbuf, pbuf, hbuf, hacc, sem):
    c = lax.axis_index("c")
    s = lax.axis_index("s")

    one16 = jnp.where(lax.iota(jnp.int32, 16) == 0, 1.0, 0.0)
    zero16 = jnp.zeros((16,), jnp.float32)

    def _z(e, carry):
        for q in range(ACC_W // 16):
            hbuf[e, pl.ds(q * 16, 16)] = zero16
        return carry
    lax.fori_loop(0, CHUNK, _z, 0)
    row0 = s * ROWS_PER_TILE
    for k in range(5):
        sz = CHUNK if k < 4 else ROWS_PER_TILE - 4 * CHUNK
        pltpu.sync_copy(hbuf.at[pl.ds(0, sz)],
                        hacc.at[pl.ds(row0 + k * CHUNK, sz)])
    plsc.subcore_barrier()

    pltpu.sync_copy(src_hbm.at[pl.ds(s * NCHUNK_TILE, NCHUNK_TILE)], src_idx)
    pltpu.sync_copy(dst_hbm.at[pl.ds(s * NCHUNK_TILE, NCHUNK_TILE)], dst_idx)

    def _chunk(j, carry):
        pltpu.async_copy(g_hbm.at[c].at[src_idx.at[j]], gbuf, sem).wait()
        pltpu.sync_copy(
            p_hbm.at[c, pl.ds((s * NCHUNK_TILE + j) * CHUNK, CHUNK)], pbuf)

        def _edge(e, inner):
            for q in range(GW // 16):
                g = gbuf[e, pl.ds(q * 16, 16)]
                p = pbuf[e, pl.ds(q * 16, 16)]
                hbuf[e, pl.ds(q * 16, 16)] = jnp.maximum(g + p, 0.0)
            hbuf[e, pl.ds(GW, 16)] = one16
            return inner
        lax.fori_loop(0, CHUNK, _edge, 0)

        pltpu.sync_copy(hbuf, hacc.at[dst_idx.at[j]], add=True)
        return carry
    lax.fori_loop(0, NCHUNK_TILE, _chunk, 0)
    plsc.subcore_barrier()

    for k in range(5):
        sz = CHUNK if k < 4 else ROWS_PER_TILE - 4 * CHUNK
        pltpu.sync_copy(hacc.at[pl.ds(row0 + k * CHUNK, sz)],
                        hbuf.at[pl.ds(0, sz)])
        pltpu.sync_copy(hbuf.at[pl.ds(0, sz)],
                        out_hbm.at[c, pl.ds(row0 + k * CHUNK, sz)])


def _node_kernel(x_ref, h0_ref, h1_ref, w2_ref, b2_ref, w3_ref, b3_ref,
                 w4_ref, b4_ref, o_ref):
    h0 = h0_ref[0]
    h1 = h1_ref[0]
    hs = jnp.concatenate([h0[:, 0:GW], h1[:, 0:GW]], axis=1)
    deg = h0[:, GW:GW + 1]
    agg = (jnp.dot(hs, w2_ref[...], preferred_element_type=jnp.float32)
           + deg * b2_ref[...])
    a = (jnp.dot(x_ref[...], w3_ref[0:D, :], preferred_element_type=jnp.float32)
         + jnp.dot(agg, w3_ref[D:2 * D, :], preferred_element_type=jnp.float32)
         + b3_ref[...])
    u = jnp.maximum(a, 0.0)
    o_ref[...] = (jnp.dot(u, w4_ref[...], preferred_element_type=jnp.float32)
                  + b4_ref[...])


def kernel(x, edge_index, edge_pe, W1, b1, W2, b2, W3, b3, W4, b4):
    src = jnp.asarray(edge_index[0], jnp.int32)
    dst = jnp.asarray(edge_index[1], jnp.int32)
    npad = E_PAD - E
    src2d = jnp.concatenate(
        [src, jnp.zeros((npad,), jnp.int32)]).reshape(E_PAD // CHUNK, CHUNK)
    dst2d = jnp.concatenate(
        [dst, jnp.full((npad,), N_NODES, jnp.int32)]).reshape(E_PAD // CHUNK, CHUNK)
    pe_pad = jnp.concatenate(
        [edge_pe, jnp.zeros((npad, PE_DIM), jnp.float32)], axis=0)
    w1a = jnp.stack([W1[:D, :GW], W1[:D, GW:]])
    w1b = jnp.stack([W1[D:, :GW], W1[D:, GW:]])
    b1h = jnp.stack([b1[:GW].reshape(1, GW), b1[GW:].reshape(1, GW)])
    b2r = b2.reshape(1, D)
    b3r = b3.reshape(1, D)
    b4r = b4.reshape(1, D)

    g_table = pl.pallas_call(
        _g_kernel,
        grid=(2,),
        in_specs=[pl.BlockSpec((N_NODES, D), lambda i: (0, 0)),
                  pl.BlockSpec((1, D, GW), lambda i: (i, 0, 0))],
        out_specs=pl.BlockSpec((1, N_NODES, GW), lambda i: (i, 0, 0)),
        out_shape=jax.ShapeDtypeStruct((2, N_NODES, GW), jnp.float32),
    )(x, w1a)

    PB = 10240
    p_edges = pl.pallas_call(
        _p_kernel,
        grid=(2, E_PAD // PB),
        in_specs=[pl.BlockSpec((PB, PE_DIM), lambda i, j: (j, 0)),
                  pl.BlockSpec((1, PE_DIM, GW), lambda i, j: (i, 0, 0)),
                  pl.BlockSpec((1, 1, GW), lambda i, j: (i, 0, 0))],
        out_specs=pl.BlockSpec((1, PB, GW), lambda i, j: (i, j, 0)),
        out_shape=jax.ShapeDtypeStruct((2, E_PAD, GW), jnp.float32),
    )(pe_pad, w1b, b1h)

    mesh = plsc.VectorSubcoreMesh(core_axis_name="c", subcore_axis_name="s")
    hacc = pl.kernel(
        _edge_kernel,
        mesh=mesh,
        compiler_params=pltpu.CompilerParams(use_tc_tiling_on_sc=False),
        out_type=jax.ShapeDtypeStruct((2, H_ROWS, ACC_W), jnp.float32),
        scratch_types=[
            pltpu.VMEM((NCHUNK_TILE, CHUNK), jnp.int32),
            pltpu.VMEM((NCHUNK_TILE, CHUNK), jnp.int32),
            pltpu.VMEM((CHUNK, GW), jnp.float32),
            pltpu.VMEM((CHUNK, GW), jnp.float32),
            pltpu.VMEM((CHUNK, ACC_W), jnp.float32),
            pltpu.VMEM_SHARED((H_ROWS, ACC_W), jnp.float32),
            pltpu.SemaphoreType.DMA,
        ],
    )(g_table, p_edges, src2d, dst2d)

    NB = 1000
    out = pl.pallas_call(
        _node_kernel,
        grid=(N_NODES // NB,),
        in_specs=[pl.BlockSpec((NB, D), lambda i: (i, 0)),
                  pl.BlockSpec((1, NB, ACC_W), lambda i: (0, i, 0)),
                  pl.BlockSpec((1, NB, ACC_W), lambda i: (1, i, 0)),
                  pl.BlockSpec((D, D), lambda i: (0, 0)),
                  pl.BlockSpec((1, D), lambda i: (0, 0)),
                  pl.BlockSpec((2 * D, D), lambda i: (0, 0)),
                  pl.BlockSpec((1, D), lambda i: (0, 0)),
                  pl.BlockSpec((D, D), lambda i: (0, 0)),
                  pl.BlockSpec((1, D), lambda i: (0, 0))],
        out_specs=pl.BlockSpec((NB, D), lambda i: (i, 0)),
        out_shape=jax.ShapeDtypeStruct((N_NODES, D), jnp.float32),
    )(x, hacc, hacc, W2, b2r, W3, b3r, W4, b4r)
    return out


# speedup vs baseline: 1.6986x; 1.3482x over previous
import jax
import jax.numpy as jnp
from jax import lax
from jax.experimental import pallas as pl
from jax.experimental.pallas import tpu as pltpu
from jax.experimental.pallas import tpu_sc as plsc

N_NODES = 10000
E = 320000
D = 128
PE_DIM = 16
GW = 64
ACC_W = 80
H_ROWS = 10112
ROWS_PER_TILE = H_ROWS // 16
CHUNK = 64
E_PAD = 327680
NCHUNK_TILE = E_PAD // CHUNK // 16
N_ACC_CP = -(-ROWS_PER_TILE // CHUNK)


def _g_kernel(x_ref, w1a_ref, o_ref):
    o_ref[0] = jnp.dot(x_ref[...], w1a_ref[0],
                       preferred_element_type=jnp.float32)


def _p_kernel(pe_ref, w1b_ref, b1_ref, o_ref):
    o_ref[0] = jnp.dot(pe_ref[...], w1b_ref[0],
                       preferred_element_type=jnp.float32) + b1_ref[0]


def _edge_kernel(g_hbm, p_hbm, src_hbm, dst_hbm, out_hbm,
                 src_idx, dst_idx, gbuf, pbuf, hbuf,
                 hacc, gsem0, gsem1, psem0, psem1, ssem0, ssem1):
    c = lax.axis_index("c")
    s = lax.axis_index("s")
    gsem = (gsem0, gsem1)
    psem = (psem0, psem1)
    ssem = (ssem0, ssem1)

    one16 = jnp.where(lax.iota(jnp.int32, 16) == 0, 1.0, 0.0)
    zero16 = jnp.zeros((16,), jnp.float32)

    def _z(e, carry):
        for b in range(2):
            for q in range(ACC_W // 16):
                hbuf[b, e, pl.ds(q * 16, 16)] = zero16
        return carry
    lax.fori_loop(0, CHUNK, _z, 0)
    row0 = s * ROWS_PER_TILE
    for k in range(N_ACC_CP):
        sz = min(CHUNK, ROWS_PER_TILE - k * CHUNK)
        pltpu.sync_copy(hbuf.at[0, pl.ds(0, sz)],
                        hacc.at[pl.ds(row0 + k * CHUNK, sz)])

    def _o(e, carry):
        for b in range(2):
            hbuf[b, e, pl.ds(GW, 16)] = one16
        return carry
    lax.fori_loop(0, CHUNK, _o, 0)
    plsc.subcore_barrier()

    pltpu.sync_copy(src_hbm.at[pl.ds(s * NCHUNK_TILE, NCHUNK_TILE)], src_idx)
    pltpu.sync_copy(dst_hbm.at[pl.ds(s * NCHUNK_TILE, NCHUNK_TILE)], dst_idx)

    def _in_refs(j, b):
        return (
            (g_hbm.at[c].at[src_idx.at[j]], gbuf.at[b], gsem[b]),
            (p_hbm.at[c, pl.ds((s * NCHUNK_TILE + j) * CHUNK, CHUNK)],
             pbuf.at[b], psem[b]),
        )

    def _start_in(j, b):
        for refs in _in_refs(j, b):
            pltpu.async_copy(*refs)

    def _wait_in(j, b):
        for refs in _in_refs(j, b):
            pltpu.make_async_copy(*refs).wait()

    def _start_scatter(j, b):
        pltpu.async_copy(hbuf.at[b], hacc.at[dst_idx.at[j]], ssem[b],
                         add=True)

    def _wait_scatter(j, b):
        pltpu.make_async_copy(hbuf.at[b], hacc.at[dst_idx.at[j]],
                              ssem[b]).wait()

    _start_in(0, 0)

    def _step(j, b):
        _wait_in(j, b)

        @pl.when(j + 1 < NCHUNK_TILE)
        def _prefetch():
            _start_in(j + 1, 1 - b)

        @pl.when(j >= 2)
        def _drain():
            _wait_scatter(j - 2, b)

        def _edge(e, inner):
            for q in range(GW // 16):
                g = gbuf[b, e, pl.ds(q * 16, 16)]
                p = pbuf[b, e, pl.ds(q * 16, 16)]
                hbuf[b, e, pl.ds(q * 16, 16)] = jnp.maximum(g + p, 0.0)
            return inner
        lax.fori_loop(0, CHUNK, _edge, 0, unroll=2)

        _start_scatter(j, b)

    def _pair(t, carry):
        _step(2 * t, 0)
        _step(2 * t + 1, 1)
        return carry
    lax.fori_loop(0, NCHUNK_TILE // 2, _pair, 0)
    _wait_scatter(NCHUNK_TILE - 2, 0)
    _wait_scatter(NCHUNK_TILE - 1, 1)
    plsc.subcore_barrier()

    for k in range(N_ACC_CP):
        sz = min(CHUNK, ROWS_PER_TILE - k * CHUNK)
        pltpu.sync_copy(hacc.at[pl.ds(row0 + k * CHUNK, sz)],
                        hbuf.at[0, pl.ds(0, sz)])
        pltpu.sync_copy(hbuf.at[0, pl.ds(0, sz)],
                        out_hbm.at[c, pl.ds(row0 + k * CHUNK, sz)])


def _node_kernel(x_ref, h0_ref, h1_ref, w2_ref, b2_ref, w3_ref, b3_ref,
                 w4_ref, b4_ref, o_ref):
    h0 = h0_ref[0]
    h1 = h1_ref[0]
    hs = jnp.concatenate([h0[:, 0:GW], h1[:, 0:GW]], axis=1)
    deg = h0[:, GW:GW + 1]
    agg = (jnp.dot(hs, w2_ref[...], preferred_element_type=jnp.float32)
           + deg * b2_ref[...])
    a = (jnp.dot(x_ref[...], w3_ref[0:D, :], preferred_element_type=jnp.float32)
         + jnp.dot(agg, w3_ref[D:2 * D, :], preferred_element_type=jnp.float32)
         + b3_ref[...])
    u = jnp.maximum(a, 0.0)
    o_ref[...] = (jnp.dot(u, w4_ref[...], preferred_element_type=jnp.float32)
                  + b4_ref[...])


def kernel(x, edge_index, edge_pe, W1, b1, W2, b2, W3, b3, W4, b4):
    src = jnp.asarray(edge_index[0], jnp.int32)
    dst = jnp.asarray(edge_index[1], jnp.int32)
    npad = E_PAD - E
    src2d = jnp.concatenate(
        [src, jnp.zeros((npad,), jnp.int32)]).reshape(E_PAD // CHUNK, CHUNK)
    dst2d = jnp.concatenate(
        [dst, jnp.full((npad,), N_NODES, jnp.int32)]).reshape(E_PAD // CHUNK, CHUNK)
    pe_pad = jnp.concatenate(
        [edge_pe, jnp.zeros((npad, PE_DIM), jnp.float32)], axis=0)
    w1a = jnp.stack([W1[:D, :GW], W1[:D, GW:]])
    w1b = jnp.stack([W1[D:, :GW], W1[D:, GW:]])
    b1h = jnp.stack([b1[:GW].reshape(1, GW), b1[GW:].reshape(1, GW)])
    b2r = b2.reshape(1, D)
    b3r = b3.reshape(1, D)
    b4r = b4.reshape(1, D)

    g_table = pl.pallas_call(
        _g_kernel,
        grid=(2,),
        in_specs=[pl.BlockSpec((N_NODES, D), lambda i: (0, 0)),
                  pl.BlockSpec((1, D, GW), lambda i: (i, 0, 0))],
        out_specs=pl.BlockSpec((1, N_NODES, GW), lambda i: (i, 0, 0)),
        out_shape=jax.ShapeDtypeStruct((2, N_NODES, GW), jnp.float32),
    )(x, w1a)

    PB = 10240
    p_edges = pl.pallas_call(
        _p_kernel,
        grid=(2, E_PAD // PB),
        in_specs=[pl.BlockSpec((PB, PE_DIM), lambda i, j: (j, 0)),
                  pl.BlockSpec((1, PE_DIM, GW), lambda i, j: (i, 0, 0)),
                  pl.BlockSpec((1, 1, GW), lambda i, j: (i, 0, 0))],
        out_specs=pl.BlockSpec((1, PB, GW), lambda i, j: (i, j, 0)),
        out_shape=jax.ShapeDtypeStruct((2, E_PAD, GW), jnp.float32),
    )(pe_pad, w1b, b1h)

    mesh = plsc.VectorSubcoreMesh(core_axis_name="c", subcore_axis_name="s")
    hacc = pl.kernel(
        _edge_kernel,
        mesh=mesh,
        compiler_params=pltpu.CompilerParams(use_tc_tiling_on_sc=False),
        out_type=jax.ShapeDtypeStruct((2, H_ROWS, ACC_W), jnp.float32),
        scratch_types=[
            pltpu.VMEM((NCHUNK_TILE, CHUNK), jnp.int32),
            pltpu.VMEM((NCHUNK_TILE, CHUNK), jnp.int32),
            pltpu.VMEM((2, CHUNK, GW), jnp.float32),
            pltpu.VMEM((2, CHUNK, GW), jnp.float32),
            pltpu.VMEM((2, CHUNK, ACC_W), jnp.float32),
            pltpu.VMEM_SHARED((H_ROWS, ACC_W), jnp.float32),
            pltpu.SemaphoreType.DMA,
            pltpu.SemaphoreType.DMA,
            pltpu.SemaphoreType.DMA,
            pltpu.SemaphoreType.DMA,
            pltpu.SemaphoreType.DMA,
            pltpu.SemaphoreType.DMA,
        ],
    )(g_table, p_edges, src2d, dst2d)

    NB = 1000
    out = pl.pallas_call(
        _node_kernel,
        grid=(N_NODES // NB,),
        in_specs=[pl.BlockSpec((NB, D), lambda i: (i, 0)),
                  pl.BlockSpec((1, NB, ACC_W), lambda i: (0, i, 0)),
                  pl.BlockSpec((1, NB, ACC_W), lambda i: (1, i, 0)),
                  pl.BlockSpec((D, D), lambda i: (0, 0)),
                  pl.BlockSpec((1, D), lambda i: (0, 0)),
                  pl.BlockSpec((2 * D, D), lambda i: (0, 0)),
                  pl.BlockSpec((1, D), lambda i: (0, 0)),
                  pl.BlockSpec((D, D), lambda i: (0, 0)),
                  pl.BlockSpec((1, D), lambda i: (0, 0))],
        out_specs=pl.BlockSpec((NB, D), lambda i: (i, 0)),
        out_shape=jax.ShapeDtypeStruct((N_NODES, D), jnp.float32),
    )(x, hacc, hacc, W2, b2r, W3, b3r, W4, b4r)
    return out
